# Initial kernel scaffold; baseline (speedup 1.0000x reference)
import jax
import jax.numpy as jnp
from jax.experimental import pallas as pl


def kernel(input, edge_index, edge_weight, W, b):
    raise NotImplementedError("write your pallas kernel here")


# speedup vs baseline: 2.2938x; 2.2938x over previous
import functools

import jax
import jax.numpy as jnp
from jax import lax
from jax.experimental import pallas as pl
from jax.experimental.pallas import tpu as pltpu
from jax.experimental.pallas import tpu_sc as plsc

_LANES = 16
_NC = 2
_NS = 16
_NW = _NC * _NS
_K = 128


def _matmul_block(x_ref, w_ref, b_ref, out_ref):
    out_ref[...] = lax.dot_general(
        x_ref[...], w_ref[...], (((1,), (1,)), ((), ())),
        preferred_element_type=jnp.float32) + b_ref[...]


def _add_block(p_ref, out_ref):
    out_ref[...] = p_ref[0] + p_ref[1]


def _make_scatter(n_pad, d, ep):
    per_tile = ep // _NW
    blocks = per_tile // _K
    rows_per_tile = n_pad // _NS
    zc = _K
    mesh = plsc.VectorSubcoreMesh(core_axis_name="c", subcore_axis_name="s")

    nsub = d // _LANES
    k8 = _K * nsub
    sub_rows = n_pad * nsub
    sub_per_tile = sub_rows // _NS

    @functools.partial(
        pl.kernel,
        out_type=jax.ShapeDtypeStruct((_NC, n_pad, d), jnp.float32),
        mesh=mesh,
        compiler_params=pltpu.CompilerParams(use_tc_tiling_on_sc=False),
        scratch_types=[
            pltpu.VMEM((_K,), jnp.int32),
            pltpu.VMEM((k8,), jnp.int32),
            pltpu.VMEM((_K * _LANES,), jnp.float32),
            pltpu.VMEM((_K, d), jnp.float32),
            pltpu.VMEM((k8, _LANES), jnp.float32),
            pltpu.VMEM_SHARED((sub_rows, _LANES), jnp.float32),
            pltpu.SemaphoreType.DMA,
        ],
    )
    def scatter(support_hbm, src_hbm, dst8_hbm, w_hbm, out_hbm,
                src_v, dst8_v, w_v, rows_v, rows8_v, acc, sem):
        cid = lax.axis_index("c")
        sid = lax.axis_index("s")
        ebase = (cid * _NS + sid) * per_tile
        rbase = sid * sub_per_tile

        def zero_row(r, carry):
            rows8_v[r, :] = jnp.zeros((_LANES,), jnp.float32)
            return carry
        lax.fori_loop(0, k8, zero_row, 0)
        for i in range(sub_per_tile // k8):
            pltpu.sync_copy(rows8_v.at[pl.ds(0, k8)],
                            acc.at[pl.ds(rbase + i * k8, k8)])
        plsc.subcore_barrier()

        def block_body(blk, carry):
            eb = pl.multiple_of(ebase + blk * _K, _K)
            pltpu.sync_copy(src_hbm.at[pl.ds(eb, _K)], src_v)
            pltpu.sync_copy(
                dst8_hbm.at[pl.ds(pl.multiple_of(eb * nsub, k8), k8)], dst8_v)
            pltpu.sync_copy(
                w_hbm.at[pl.ds(pl.multiple_of(eb * _LANES, _K * _LANES),
                               _K * _LANES)], w_v)
            pltpu.async_copy(support_hbm.at[src_v], rows_v, sem).wait()

            def edge_body(e, c2):
                wv = w_v[pl.ds(pl.multiple_of(e * _LANES, _LANES), _LANES)]
                e8 = e * nsub
                for j in range(nsub):
                    rows8_v[e8 + j, :] = (
                        rows_v[e, pl.ds(j * _LANES, _LANES)] * wv)
                return c2
            lax.fori_loop(0, _K, edge_body, 0)

            pltpu.sync_copy(rows8_v, acc.at[dst8_v], add=True)
            return carry
        lax.fori_loop(0, blocks, block_body, 0)

        plsc.subcore_barrier()
        rowbase = sid * (n_pad // _NS)
        for i in range(sub_per_tile // k8):
            pltpu.sync_copy(acc.at[pl.ds(rbase + i * k8, k8)], rows8_v)

            def repack_row(r, carry):
                r8 = r * nsub
                for j in range(nsub):
                    rows_v[r, pl.ds(j * _LANES, _LANES)] = rows8_v[r8 + j, :]
                return carry
            lax.fori_loop(0, _K, repack_row, 0)
            pltpu.sync_copy(
                rows_v, out_hbm.at[cid, pl.ds(rowbase + i * _K, _K)])

    return scatter


def kernel(input, edge_index, edge_weight, W, b):
    n, d_in = input.shape
    d_out = W.shape[0]
    e = edge_weight.shape[0]
    assert d_in % _LANES == 0 and d_out % _LANES == 0
    n_pad = -(-n // (_NS * _K)) * (_NS * _K)

    rb = 1000
    grid = (n // rb,)
    support = pl.pallas_call(
        _matmul_block,
        grid=grid,
        in_specs=[pl.BlockSpec((rb, d_in), lambda i: (i, 0)),
                  pl.BlockSpec((d_out, d_in), lambda i: (0, 0)),
                  pl.BlockSpec((1, d_out), lambda i: (0, 0))],
        out_specs=pl.BlockSpec((rb, d_out), lambda i: (i, 0)),
        out_shape=jax.ShapeDtypeStruct((n, d_out), jnp.float32),
    )(input, W, b.reshape(1, d_out))

    chunk = _NW * _K
    ep = ((e + chunk - 1) // chunk) * chunk
    pad = ep - e
    src = jnp.concatenate(
        [edge_index[1].astype(jnp.int32), jnp.zeros((pad,), jnp.int32)])
    dst = jnp.concatenate(
        [edge_index[0].astype(jnp.int32), jnp.zeros((pad,), jnp.int32)])
    w = jnp.concatenate(
        [edge_weight.astype(jnp.float32), jnp.zeros((pad,), jnp.float32)])
    w = jnp.broadcast_to(w[:, None], (ep, _LANES)).reshape(ep * _LANES)
    nsub = d_out // _LANES
    dst8 = (dst[:, None] * nsub + jnp.arange(nsub, dtype=jnp.int32)
            ).reshape(ep * nsub)

    partials = _make_scatter(n_pad, d_out, ep)(support, src, dst8, w)

    out = pl.pallas_call(
        _add_block,
        grid=grid,
        in_specs=[pl.BlockSpec((_NC, rb, d_out), lambda i: (0, i, 0))],
        out_specs=pl.BlockSpec((rb, d_out), lambda i: (i, 0)),
        out_shape=jax.ShapeDtypeStruct((n, d_out), jnp.float32),
    )(partials)
    return out
